# baseline (speedup 1.0000x reference)
import functools

import jax
import jax.numpy as jnp
from jax import lax
from jax.experimental import pallas as pl
from jax.experimental.pallas import tpu as pltpu
from jax.experimental.pallas import tpu_sc as plsc

D_MODEL = 1024
NUM_WORKERS = 32
B_PER_W = (4 * 4096) // NUM_WORKERS
CHUNK = 16
NBUF = 4
NGROUPS = B_PER_W // (CHUNK * NBUF)

_mesh = plsc.VectorSubcoreMesh(core_axis_name="c", subcore_axis_name="s")

_scratch = (
    [pltpu.VMEM((B_PER_W,), jnp.int32)]
    + [pltpu.VMEM((CHUNK, D_MODEL), jnp.float32) for _ in range(NBUF)]
    + [pltpu.SemaphoreType.DMA for _ in range(2 * NBUF)]
)


@functools.partial(
    pl.kernel,
    out_type=jax.ShapeDtypeStruct((4, 4096, D_MODEL), jnp.float32),
    mesh=_mesh,
    scratch_types=_scratch,
)
def _embed_sc(x_hbm, table_hbm, out_hbm, idx_v, *bufs_and_sems):
    bufs = bufs_and_sems[:NBUF]
    gsems = bufs_and_sems[NBUF:2 * NBUF]
    ssems = bufs_and_sems[2 * NBUF:]

    wid = lax.axis_index("s") * 2 + lax.axis_index("c")
    row = wid // 8
    col0 = (wid % 8) * B_PER_W
    pltpu.sync_copy(x_hbm.at[row, pl.ds(col0, B_PER_W)], idx_v)

    def gather(flat_chunk_base, b):
        return pltpu.async_copy(
            table_hbm.at[idx_v.at[pl.ds(flat_chunk_base + b * CHUNK, CHUNK)]],
            bufs[b], gsems[b])

    def wait_gather(b):
        pltpu.make_async_copy(
            table_hbm.at[pl.ds(0, CHUNK)], bufs[b], gsems[b]).wait()

    def scatter(flat_chunk_base, b):
        return pltpu.async_copy(
            bufs[b],
            out_hbm.at[row, pl.ds(col0 + flat_chunk_base + b * CHUNK, CHUNK)],
            ssems[b])

    def wait_scatter(b):
        pltpu.make_async_copy(
            bufs[b], out_hbm.at[row, pl.ds(col0, CHUNK)], ssems[b]).wait()

    for b in range(NBUF):
        gather(0, b)

    def body(o, carry):
        base = o * NBUF * CHUNK
        nxt_base = base + NBUF * CHUNK
        for b in range(NBUF):
            wait_gather(b)
            scatter(base, b)

        @pl.when(o + 1 < NGROUPS)
        def _():
            for b in range(NBUF):
                wait_scatter(b)
                gather(nxt_base, b)

        return carry

    lax.fori_loop(0, NGROUPS, body, 0)
    for b in range(NBUF):
        wait_scatter(b)


def kernel(x, table):
    return _embed_sc(x.astype(jnp.int32), table)


# speedup vs baseline: 1.0490x; 1.0490x over previous
import functools

import jax
import jax.numpy as jnp
from jax import lax
from jax.experimental import pallas as pl
from jax.experimental.pallas import tpu as pltpu
from jax.experimental.pallas import tpu_sc as plsc

D_MODEL = 1024
NUM_WORKERS = 32
B_PER_W = (4 * 4096) // NUM_WORKERS
CHUNK = 16
NBUF = 6
NCHUNKS = B_PER_W // CHUNK

_mesh = plsc.VectorSubcoreMesh(core_axis_name="c", subcore_axis_name="s")

_scratch = (
    [pltpu.VMEM((B_PER_W,), jnp.int32)]
    + [pltpu.VMEM((CHUNK, D_MODEL), jnp.float32) for _ in range(NBUF)]
    + [pltpu.SemaphoreType.DMA for _ in range(2 * NBUF)]
)


@functools.partial(
    pl.kernel,
    out_type=jax.ShapeDtypeStruct((4, 4096, D_MODEL), jnp.float32),
    mesh=_mesh,
    scratch_types=_scratch,
)
def _embed_sc(x_hbm, table_hbm, out_hbm, idx_v, *bufs_and_sems):
    bufs = bufs_and_sems[:NBUF]
    gsems = bufs_and_sems[NBUF:2 * NBUF]
    ssems = bufs_and_sems[2 * NBUF:]

    wid = lax.axis_index("s") * 2 + lax.axis_index("c")
    row = wid // 8
    col0 = (wid % 8) * B_PER_W
    pltpu.sync_copy(x_hbm.at[row, pl.ds(col0, B_PER_W)], idx_v)

    gathers = [None] * NCHUNKS
    scatters = [None] * NCHUNKS
    for g in range(min(NBUF, NCHUNKS)):
        gathers[g] = pltpu.async_copy(
            table_hbm.at[idx_v.at[pl.ds(g * CHUNK, CHUNK)]], bufs[g], gsems[g])
    for g in range(NCHUNKS):
        cur = g % NBUF
        gathers[g].wait()
        scatters[g] = pltpu.async_copy(
            bufs[cur],
            out_hbm.at[row, pl.ds(col0 + g * CHUNK, CHUNK)], ssems[cur])
        if g + NBUF < NCHUNKS:
            scatters[g].wait()
            gathers[g + NBUF] = pltpu.async_copy(
                table_hbm.at[idx_v.at[pl.ds((g + NBUF) * CHUNK, CHUNK)]],
                bufs[cur], gsems[cur])
    for g in range(max(0, NCHUNKS - NBUF), NCHUNKS):
        scatters[g].wait()


def kernel(x, table):
    return _embed_sc(x.astype(jnp.int32), table)
